# Initial kernel scaffold; baseline (speedup 1.0000x reference)
import jax
import jax.numpy as jnp
from jax.experimental import pallas as pl


def kernel(adj, inputs, W):
    raise NotImplementedError("write your pallas kernel here")


# speedup vs baseline: 1.0365x; 1.0365x over previous
import functools

import jax
import jax.numpy as jnp
from jax.experimental import pallas as pl
from jax.experimental.pallas import tpu as pltpu


def _body(adj_ref, x_ref, w_ref, out_ref, xw_ref):
    @pl.when(pl.program_id(0) == 0)
    def _():
        xw_ref[:] = jnp.dot(x_ref[:], w_ref[:],
                            preferred_element_type=jnp.float32)

    acc = jnp.dot(adj_ref[:], xw_ref[:],
                  preferred_element_type=jnp.float32)
    out_ref[:] = jnp.maximum(acc, 0.0)


@functools.partial(jax.jit, static_argnames=("block_rows",))
def _gcn(adj, inputs, W, block_rows):
    n, _ = adj.shape
    d_out = W.shape[1]
    grid = (n // block_rows,)
    return pl.pallas_call(
        _body,
        grid=grid,
        in_specs=[
            pl.BlockSpec((block_rows, n), lambda i: (i, 0)),
            pl.BlockSpec(inputs.shape, lambda i: (0, 0)),
            pl.BlockSpec(W.shape, lambda i: (0, 0)),
        ],
        out_specs=pl.BlockSpec((block_rows, d_out), lambda i: (i, 0)),
        out_shape=jax.ShapeDtypeStruct((n, d_out), jnp.float32),
        scratch_shapes=[pltpu.VMEM((inputs.shape[0], d_out), jnp.float32)],
    )(adj, inputs, W)


def kernel(adj, inputs, W):
    n = adj.shape[0]
    block_rows = 8
    for b in range(8, 512, 8):
        if n % b == 0:
            block_rows = b
    return _gcn(adj, inputs, W, block_rows)


# speedup vs baseline: 1.0382x; 1.0016x over previous
import functools

import jax
import jax.numpy as jnp
from jax.experimental import pallas as pl
from jax.experimental.pallas import tpu as pltpu


def _body(adj_ref, x_ref, w_ref, out_ref, xw_ref):
    @pl.when(pl.program_id(0) == 0)
    def _():
        xw_ref[:] = jnp.dot(x_ref[:], w_ref[:],
                            preferred_element_type=jnp.float32)

    acc = jnp.dot(adj_ref[:].astype(jnp.bfloat16),
                  xw_ref[:].astype(jnp.bfloat16),
                  preferred_element_type=jnp.float32)
    out_ref[:] = jnp.maximum(acc, 0.0)


@functools.partial(jax.jit, static_argnames=("block_rows",))
def _gcn(adj, inputs, W, block_rows):
    n, _ = adj.shape
    d_out = W.shape[1]
    grid = (n // block_rows,)
    return pl.pallas_call(
        _body,
        grid=grid,
        in_specs=[
            pl.BlockSpec((block_rows, n), lambda i: (i, 0)),
            pl.BlockSpec(inputs.shape, lambda i: (0, 0)),
            pl.BlockSpec(W.shape, lambda i: (0, 0)),
        ],
        out_specs=pl.BlockSpec((block_rows, d_out), lambda i: (i, 0)),
        out_shape=jax.ShapeDtypeStruct((n, d_out), jnp.float32),
        scratch_shapes=[pltpu.VMEM((inputs.shape[0], d_out), jnp.float32)],
    )(adj, inputs, W)


def kernel(adj, inputs, W):
    n = adj.shape[0]
    block_rows = 8
    for b in range(8, 512, 8):
        if n % b == 0:
            block_rows = b
    return _gcn(adj, inputs, W, block_rows)
